# baseline (speedup 1.0000x reference)
import functools

import jax
import jax.numpy as jnp
from jax import lax
from jax.experimental import pallas as pl
from jax.experimental.pallas import tpu as pltpu
from jax.experimental.pallas import tpu_sc as plsc

N_NODES = 10000
N_EDGES = 320000
H = 128
HW = H // 2
L = 16
KV = HW // L
NW = 32
EPW = N_EDGES // NW
CH = 80
NCHUNK = EPW // CH
NBUF = 2


def _tables_body(h_ref, wa_ref, wb_ref, b1_ref, a_ref, b_ref):
    x = h_ref[...]
    a_ref[...] = (
        jnp.dot(x, wa_ref[...], preferred_element_type=jnp.float32) + b1_ref[...]
    )
    b_ref[...] = jnp.dot(x, wb_ref[...], preferred_element_type=jnp.float32)


def _node_tables(h, W1, b1):
    rows = 1000
    grid = (N_NODES // rows,)
    return pl.pallas_call(
        _tables_body,
        grid=grid,
        in_specs=[
            pl.BlockSpec((rows, H), lambda i: (i, 0)),
            pl.BlockSpec((H, H), lambda i: (0, 0)),
            pl.BlockSpec((H, H), lambda i: (0, 0)),
            pl.BlockSpec((1, H), lambda i: (0, 0)),
        ],
        out_specs=[
            pl.BlockSpec((rows, H), lambda i: (i, 0)),
            pl.BlockSpec((rows, H), lambda i: (i, 0)),
        ],
        out_shape=[
            jax.ShapeDtypeStruct((N_NODES, H), jnp.float32),
            jax.ShapeDtypeStruct((N_NODES, H), jnp.float32),
        ],
    )(h, W1[:H], W1[H:], b1.reshape(1, H))


def _pack_bf16_pairs(t):
    u = lax.bitcast_convert_type(t.astype(jnp.bfloat16), jnp.uint16).astype(jnp.uint32)
    w = (u[:, :HW] << 16) | u[:, HW:]
    return lax.bitcast_convert_type(w, jnp.int32)


def _permute(a, perm):
    return lax.gather(
        a, perm[:, None],
        lax.GatherDimensionNumbers(
            offset_dims=(), collapsed_slice_dims=(0,), start_index_map=(0,)
        ),
        slice_sizes=(1,),
        mode=lax.GatherScatterMode.PROMISE_IN_BOUNDS,
        unique_indices=True, indices_are_sorted=False,
    )


@functools.partial(
    pl.kernel,
    out_type=jax.ShapeDtypeStruct((N_EDGES,), jnp.float32),
    mesh=plsc.VectorSubcoreMesh(core_axis_name="c", subcore_axis_name="s"),
    scratch_types=[
        pltpu.VMEM_SHARED((N_NODES, H), jnp.int32),
        pltpu.VMEM((NBUF, CH), jnp.int32),
        pltpu.VMEM((NBUF, CH), jnp.int32),
        pltpu.VMEM((NBUF, CH, H), jnp.int32),
        pltpu.VMEM((NBUF, CH, H), jnp.int32),
        pltpu.VMEM((H,), jnp.float32),
        pltpu.VMEM((NBUF, CH), jnp.float32),
        pltpu.SemaphoreType.DMA,
        pltpu.SemaphoreType.DMA,
    ],
)
def _edge_decode(ab_hbm, src_hbm, dst_hbm, w2_hbm, out_hbm,
                 spt, src_i, dst_i, za, zb, w2_v, out_b, semG, semO):
    sid = lax.axis_index("s")
    wid = sid * 2 + lax.axis_index("c")
    base0 = wid * EPW
    last = NCHUNK - 1

    rows_per = 1000

    @pl.when(sid < N_NODES // rows_per)
    def _():
        pltpu.sync_copy(ab_hbm.at[pl.ds(sid * rows_per, rows_per)],
                        spt.at[pl.ds(sid * rows_per, rows_per)])

    plsc.subcore_barrier()

    pltpu.sync_copy(w2_hbm, w2_v)
    w2h = [w2_v[pl.ds(k * L, L)] for k in range(KV)]
    w2l = [w2_v[pl.ds(HW + k * L, L)] for k in range(KV)]
    lane_ids = lax.iota(jnp.int32, L)
    perms = [(lane_ids + sh) & 15 for sh in (8, 4, 2, 1)]
    zero = jnp.zeros((L,), jnp.float32)

    def prefetch(c, slot):
        base = base0 + c * CH
        pltpu.sync_copy(src_hbm.at[pl.ds(base, CH)], src_i.at[slot])
        pltpu.sync_copy(dst_hbm.at[pl.ds(base, CH)], dst_i.at[slot])
        pltpu.async_copy(spt.at[src_i.at[slot]], za.at[slot], semG)
        pltpu.async_copy(spt.at[dst_i.at[slot]], zb.at[slot], semG)

    def wait_gathers(slot):
        pltpu.make_async_copy(spt.at[pl.ds(0, CH)], za.at[slot], semG).wait()
        pltpu.make_async_copy(spt.at[pl.ds(0, CH)], zb.at[slot], semG).wait()

    def wait_out(slot):
        pltpu.make_async_copy(out_b.at[slot], out_hbm.at[pl.ds(0, CH)], semO).wait()

    def compute_chunk(c, j):
        def group_body(g, gcarry):
            e0 = g * L
            red = zero
            for i in range(L):
                acc_h = zero
                acc_l = zero
                for k in range(KV):
                    ai = za[j, e0 + i, pl.ds(k * L, L)]
                    bi = zb[j, e0 + i, pl.ds(HW + k * L, L)]
                    zh = (lax.bitcast_convert_type(ai, jnp.float32)
                          + lax.bitcast_convert_type(bi, jnp.float32))
                    zl = (lax.bitcast_convert_type(ai << 16, jnp.float32)
                          + lax.bitcast_convert_type(bi << 16, jnp.float32))
                    acc_h = acc_h + jnp.maximum(zh, 0.0) * w2h[k]
                    acc_l = acc_l + jnp.maximum(zl, 0.0) * w2l[k]
                acc = acc_h + acc_l
                for p in perms:
                    acc = acc + _permute(acc, p)
                red = jnp.where(lane_ids == i, acc, red)
            out_b[j, pl.ds(e0, L)] = red
            return gcarry

        lax.fori_loop(0, CH // L, group_body, 0)
        pltpu.async_copy(out_b.at[j], out_hbm.at[pl.ds(base0 + c * CH, CH)], semO)

    prefetch(0, 0)

    def blk_body(blk, carry):
        for j in range(NBUF):
            c = blk * NBUF + j
            wait_gathers(j)
            prefetch(c + 1, (j + 1) % NBUF)

            @pl.when(c >= NBUF)
            def _():
                wait_out(j)

            compute_chunk(c, j)
        return carry

    lax.fori_loop(0, (NCHUNK - 1) // NBUF, blk_body, 0)

    wait_gathers(0)
    wait_out(0)
    compute_chunk(last, 0)

    wait_out(1)
    wait_out(0)


def kernel(edges, h, W1, b1, W2, b2):
    edges = edges.astype(jnp.int32)
    a_tab, b_tab = _node_tables(h, W1, b1)
    ab_packed = jnp.concatenate(
        [_pack_bf16_pairs(a_tab), _pack_bf16_pairs(b_tab)], axis=1
    )
    out = _edge_decode(ab_packed, edges[0], edges[1], W2.reshape(H))
    return out + b2[0]
